# Initial kernel scaffold; baseline (speedup 1.0000x reference)
import jax
import jax.numpy as jnp
from jax.experimental import pallas as pl


def kernel(x, weight):
    raise NotImplementedError("write your pallas kernel here")


# speedup vs baseline: 1.1086x; 1.1086x over previous
import functools

import jax
import jax.numpy as jnp
from jax import lax
from jax.experimental import pallas as pl
from jax.experimental.pallas import tpu as pltpu
from jax.experimental.pallas import tpu_sc as plsc

EMBED_DIM = 32
NUM_WORKERS = 32
CHUNK = 1600


def _make_gather(num_idx: int):
  assert num_idx % (NUM_WORKERS * CHUNK) == 0
  b_per_w = num_idx // NUM_WORKERS
  n_chunks = b_per_w // CHUNK
  assert n_chunks >= 2

  mesh = plsc.VectorSubcoreMesh(core_axis_name="c", subcore_axis_name="s")

  @functools.partial(
      pl.kernel,
      mesh=mesh,
      out_type=jax.ShapeDtypeStruct((num_idx, EMBED_DIM), jnp.float32),
      scratch_types=[
          pltpu.VMEM((2, CHUNK), jnp.int32),
          pltpu.VMEM((2, CHUNK, EMBED_DIM), jnp.float32),
          pltpu.SemaphoreType.DMA((2,)),
          pltpu.SemaphoreType.DMA((2,)),
          pltpu.SemaphoreType.DMA((2,)),
      ],
      compiler_params=pltpu.CompilerParams(use_tc_tiling_on_sc=False),
  )
  def gather_kernel(idx_hbm, table_hbm, out_hbm, idx_v, rows_v, isem, gsem, ssem):
    wid = lax.axis_index("s") * 2 + lax.axis_index("c")
    base = wid * b_per_w

    def idx_copy(c, slot):
      return pltpu.make_async_copy(
          idx_hbm.at[pl.ds(base + c * CHUNK, CHUNK)], idx_v.at[slot],
          isem.at[slot])

    def gather_copy(slot):
      return pltpu.make_async_copy(
          table_hbm.at[idx_v.at[slot]], rows_v.at[slot], gsem.at[slot])

    def store_copy(c, slot):
      return pltpu.make_async_copy(
          rows_v.at[slot], out_hbm.at[pl.ds(base + c * CHUNK, CHUNK)],
          ssem.at[slot])

    idx_copy(0, 0).start()
    for c in range(n_chunks):
      slot = c & 1
      idx_copy(c, slot).wait()
      if c + 1 < n_chunks:
        idx_copy(c + 1, slot ^ 1).start()
      if c >= 2:
        store_copy(c - 2, slot).wait()
      g = gather_copy(slot)
      g.start()
      g.wait()
      store_copy(c, slot).start()
    store_copy(n_chunks - 2, (n_chunks - 2) & 1).wait()
    store_copy(n_chunks - 1, (n_chunks - 1) & 1).wait()

  return gather_kernel


def kernel(x, weight):
  b, s = x.shape
  flat_idx = x.reshape(b * s).astype(jnp.int32)
  out = _make_gather(b * s)(flat_idx, weight)
  return out.reshape(b, s, EMBED_DIM)
